# baseline (speedup 1.0000x reference)
import jax
import jax.numpy as jnp
from jax import lax
from jax.experimental import pallas as pl
from jax.experimental.pallas import tpu as pltpu
from jax.experimental.pallas import tpu_sc as plsc

B = 16384
F = 26
FIELD_SIZE = 40000
NUM_WORKERS = 32
BPW = B // NUM_WORKERS
CHUNK = BPW * F
NVEC = BPW // 16
UNROLL_A = 8


def _sc_body(x_hbm, w_hbm, out_hbm, shared, s_v, idx_v, g_v, o_v, sem):
  cid = lax.axis_index("c")
  sid = lax.axis_index("s")
  wid = sid * 2 + cid
  base = wid * BPW

  pltpu.sync_copy(
      x_hbm.at[pl.ds(base * F, CHUNK)], shared.at[pl.ds(sid * CHUNK, CHUNK)]
  )

  iota = lax.iota(jnp.int32, 16)

  def build_perm(f, _):
    perm_base = sid * CHUNK + f
    for v in range(NVEC):
      s_v[pl.ds(f * BPW + v * 16, 16)] = (v * 16 + iota) * F + perm_base
    return 0

  lax.fori_loop(0, F, build_perm, 0, unroll=False)

  pltpu.async_copy(shared.at[s_v], idx_v, sem).wait()

  def add_off(f, _):
    off = f * FIELD_SIZE
    for v in range(NVEC):
      p0 = f * BPW + v * 16
      idx_v[pl.ds(p0, 16)] = idx_v[pl.ds(p0, 16)] + off
    return 0

  lax.fori_loop(0, F, add_off, 0, unroll=False)

  pltpu.async_copy(w_hbm.at[idx_v], g_v, sem).wait()

  def reduce_vec(v, _):
    acc = g_v[pl.ds(v * 16, 16)]
    for f in range(1, F):
      acc = acc + g_v[pl.ds(f * BPW + v * 16, 16)]
    o_v[pl.ds(v * 16, 16)] = acc
    return 0

  lax.fori_loop(0, NVEC, reduce_vec, 0, unroll=False)

  pltpu.sync_copy(o_v, out_hbm.at[pl.ds(base, BPW)])


@jax.jit
def _sc_lookup(x_flat, w_flat):
  mesh = plsc.VectorSubcoreMesh(
      core_axis_name="c", subcore_axis_name="s", num_cores=2, num_subcores=16
  )
  return pl.kernel(
      _sc_body,
      out_type=jax.ShapeDtypeStruct((B,), jnp.float32),
      mesh=mesh,
      scratch_types=[
          pltpu.VMEM_SHARED((16 * CHUNK,), jnp.int32),
          pltpu.VMEM((CHUNK,), jnp.int32),
          pltpu.VMEM((CHUNK,), jnp.int32),
          pltpu.VMEM((CHUNK,), jnp.float32),
          pltpu.VMEM((BPW,), jnp.float32),
          pltpu.SemaphoreType.DMA,
      ],
  )(x_flat, w_flat)


def kernel(x, fc_weight, bias):
  out = _sc_lookup(x.reshape(-1), fc_weight.reshape(-1))
  return out.reshape(B, 1) + bias[None, :]
